# baseline (speedup 1.0000x reference)
import functools

import jax
import jax.numpy as jnp
from jax import lax
from jax.experimental import pallas as pl
from jax.experimental.pallas import tpu as pltpu
from jax.experimental.pallas import tpu_sc as plsc

N = 10000
E = 320000
IN_CH = 128
HID = 256
OUT_CH = 128

NC = 2
NS = 16
NP = 10240
DUMMY = N
K = 128
CSEG = 160
CHP = 40
CES = 80
CDEG = 79
NPS = 10112
TROWS = 632
BLK = 1280
NB = NP // BLK

_WB_K = ((0, 128), (128, 128), (256, 128), (384, 128), (512, 120))
_WB_ZB = ((0, 320), (320, 312))

_mesh = plsc.VectorSubcoreMesh(core_axis_name="c", subcore_axis_name="s")


@functools.partial(
    pl.kernel,
    out_type=jax.ShapeDtypeStruct((NC, NP), jnp.float32),
    mesh=_mesh,
    compiler_params=pltpu.CompilerParams(needs_layout_passes=False),
    scratch_types=[
        pltpu.VMEM((CDEG * K,), jnp.int32),
        pltpu.VMEM((NP,), jnp.float32),
        pltpu.VMEM((NS, NP // NS), jnp.float32),
        pltpu.VMEM((NP // NS,), jnp.float32),
        pltpu.VMEM_SHARED((NS, NP), jnp.float32),
    ],
)
def _deg_kernel(dst_hbm, deg_hbm, idx_v, cnt_v, red_v, out_v, spm):
    cid = lax.axis_index("c")
    sid = lax.axis_index("s")
    rpt = NP // NS
    pltpu.sync_copy(dst_hbm.at[cid, sid], idx_v)
    zero16 = jnp.zeros((16,), jnp.float32)
    ones16 = jnp.ones((16,), jnp.float32)

    def _z(i, _):
        cnt_v[pl.ds(i * 16, 16)] = zero16
        return 0

    lax.fori_loop(0, NP // 16, _z, 0)

    def _count(i, _):
        idx16 = idx_v[pl.ds(i * 16, 16)]
        plsc.addupdate_scatter(cnt_v, [idx16], ones16)
        return 0

    lax.fori_loop(0, CDEG * K // 16, _count, 0)
    pltpu.sync_copy(cnt_v, spm.at[sid])
    plsc.subcore_barrier()
    for t in range(NS):
        pltpu.sync_copy(spm.at[t, pl.ds(sid * rpt, rpt)], red_v.at[t])

    def _red(g, _):
        acc = jnp.zeros((16,), jnp.float32)
        for t in range(NS):
            acc = acc + red_v[t, pl.ds(g * 16, 16)]
        out_v[pl.ds(g * 16, 16)] = acc
        return 0

    lax.fori_loop(0, rpt // 16, _red, 0)
    pltpu.sync_copy(out_v, deg_hbm.at[cid, pl.ds(sid * rpt, rpt)])


def _make_segsum(ch):
    @functools.partial(
        pl.kernel,
        out_type=jax.ShapeDtypeStruct((NC, NP, ch), jnp.float32),
        mesh=_mesh,
        scratch_types=[
            pltpu.VMEM((CHP, K), jnp.int32),
            pltpu.VMEM((CHP, K), jnp.int32),
            pltpu.VMEM((K, ch), jnp.float32),
            pltpu.VMEM_SHARED((NPS, ch), jnp.float32),
            pltpu.SemaphoreType.DMA,
        ],
    )
    def _segsum(p_hbm, src_hbm, dst_hbm, acc_hbm, srcv, dstv, gbuf, acc_s, sem):
        cid = lax.axis_index("c")
        sid = lax.axis_index("s")
        base = sid * TROWS

        def _fill(i, _):
            for c in range(ch // 16):
                gbuf[i, pl.ds(c * 16, 16)] = jnp.zeros((16,), jnp.float32)
            return 0

        lax.fori_loop(0, K, _fill, 0)
        for off, n in _WB_K:
            pltpu.sync_copy(gbuf.at[pl.ds(0, n)], acc_s.at[pl.ds(base + off, n)])
        plsc.subcore_barrier()

        for p0 in range(0, CSEG, CHP):
            pltpu.sync_copy(src_hbm.at[cid, sid, pl.ds(p0, CHP)], srcv)
            pltpu.sync_copy(dst_hbm.at[sid, pl.ds(p0, CHP)], dstv)

            def _chunk(j, _):
                pltpu.async_copy(p_hbm.at[srcv.at[j]], gbuf, sem).wait()
                pltpu.sync_copy(gbuf, acc_s.at[dstv.at[j]], add=True)
                return 0

            lax.fori_loop(0, CHP, _chunk, 0)

        plsc.subcore_barrier()
        for off, n in _WB_K:
            pltpu.sync_copy(acc_s.at[pl.ds(base + off, n)], gbuf.at[pl.ds(0, n)])
            pltpu.sync_copy(gbuf.at[pl.ds(0, n)], acc_hbm.at[cid, pl.ds(base + off, n)])

    return _segsum


_segsum128 = _make_segsum(128)


@functools.partial(
    pl.kernel,
    out_type=jax.ShapeDtypeStruct((NC, NP, OUT_CH), jnp.float32),
    mesh=_mesh,
    scratch_types=[
        pltpu.VMEM((CHP, K), jnp.int32),
        pltpu.VMEM((CHP, K), jnp.int32),
        pltpu.VMEM((K, OUT_CH), jnp.float32),
        pltpu.VMEM_SHARED((NPS, OUT_CH), jnp.float32),
        pltpu.SemaphoreType.DMA,
    ],
)
def _segsum_es(p_hbm, src_hbm, dst_hbm, acc_hbm, srcv, dstv, gbuf, acc_s, sem):
    cid = lax.axis_index("c")
    sid = lax.axis_index("s")
    base = sid * TROWS

    def _fill(i, _):
        for c in range(OUT_CH // 16):
            gbuf[i, pl.ds(c * 16, 16)] = jnp.zeros((16,), jnp.float32)
        return 0

    lax.fori_loop(0, K, _fill, 0)
    for off, n in _WB_K:
        pltpu.sync_copy(gbuf.at[pl.ds(0, n)], acc_s.at[pl.ds(base + off, n)])
    plsc.subcore_barrier()

    for p0 in range(0, CES, CHP):
        pltpu.sync_copy(src_hbm.at[cid, sid, pl.ds(p0, CHP)], srcv)
        pltpu.sync_copy(dst_hbm.at[cid, sid, pl.ds(p0, CHP)], dstv)

        def _chunk(j, _):
            pltpu.async_copy(p_hbm.at[srcv.at[j]], gbuf, sem).wait()
            pltpu.sync_copy(gbuf, acc_s.at[dstv.at[j]], add=True)
            return 0

        lax.fori_loop(0, CHP, _chunk, 0)

    plsc.subcore_barrier()
    for off, n in _WB_K:
        pltpu.sync_copy(acc_s.at[pl.ds(base + off, n)], gbuf.at[pl.ds(0, n)])
        pltpu.sync_copy(gbuf.at[pl.ds(0, n)], acc_hbm.at[cid, pl.ds(base + off, n)])


def _dinv_of(deg_blk):
    return lax.rsqrt(deg_blk[0, :, 0:1] + deg_blk[1, :, 0:1] + 1.0)


def _tc_a_body(deg_ref, x_ref, w_ref, p_ref):
    d = _dinv_of(deg_ref[...])
    h = jnp.dot(x_ref[...], w_ref[...], preferred_element_type=jnp.float32)
    p_ref[0] = h * d


def _tc_a(deg, xp, w1):
    return pl.pallas_call(
        _tc_a_body,
        grid=(2, NB),
        in_specs=[
            pl.BlockSpec((NC, BLK, 16), lambda ph, i: (0, i, 0)),
            pl.BlockSpec((BLK, IN_CH), lambda ph, i: (i, 0)),
            pl.BlockSpec((IN_CH, HID // 2), lambda ph, i: (0, ph)),
        ],
        out_specs=pl.BlockSpec((1, BLK, HID // 2), lambda ph, i: (ph, i, 0)),
        out_shape=jax.ShapeDtypeStruct((2, NP, HID // 2), jnp.float32),
    )(deg, xp, w1)


def _make_pre_body(ch):
    def _body(acc_ref, p_ref, deg_ref, b_ref, pre_ref, s1_ref, s2_ref):
        i = pl.program_id(0)
        d = _dinv_of(deg_ref[...])
        a = acc_ref[...]
        p = p_ref[...]
        u0 = (a[0] + p[0]) * d
        u1 = (a[1] + p[1]) * d
        pre = jnp.concatenate([u0, u1], axis=1) + b_ref[...]
        pre_ref[...] = pre
        row = i * BLK + lax.broadcasted_iota(jnp.int32, (BLK, 1), 0)
        pm = jnp.where(row < N, pre, 0.0)
        s1_ref[0, 0] = jnp.sum(pm, axis=0)
        s2_ref[0, 0] = jnp.sum(pm * pm, axis=0)

    return _body


def _tc_pre(acc, p, deg, b, ch):
    return pl.pallas_call(
        _make_pre_body(ch),
        grid=(NB,),
        in_specs=[
            pl.BlockSpec((NC, BLK, ch), lambda i: (0, i, 0)),
            pl.BlockSpec((NC, BLK, ch), lambda i: (0, i, 0)),
            pl.BlockSpec((NC, BLK, 16), lambda i: (0, i, 0)),
            pl.BlockSpec((1, 2 * ch), lambda i: (0, 0)),
        ],
        out_specs=[
            pl.BlockSpec((BLK, 2 * ch), lambda i: (i, 0)),
            pl.BlockSpec((1, 1, 2 * ch), lambda i: (i, 0, 0)),
            pl.BlockSpec((1, 1, 2 * ch), lambda i: (i, 0, 0)),
        ],
        out_shape=[
            jax.ShapeDtypeStruct((NP, 2 * ch), jnp.float32),
            jax.ShapeDtypeStruct((NB, 1, 2 * ch), jnp.float32),
            jax.ShapeDtypeStruct((NB, 1, 2 * ch), jnp.float32),
        ],
    )(acc, p, deg, b)


def _bn_stats(s1_ref, s2_ref):
    s1 = jnp.sum(s1_ref[...][:, 0, :], axis=0, keepdims=True)
    s2 = jnp.sum(s2_ref[...][:, 0, :], axis=0, keepdims=True)
    mean = s1 / N
    var = s2 / N - mean * mean
    inv = lax.rsqrt(var + 1e-5)
    return mean, inv


def _tc_b2_body(pre_ref, s1_ref, s2_ref, deg_ref, g_ref, be_ref, w_ref, p2_ref):
    mean, inv = _bn_stats(s1_ref, s2_ref)
    bn = (pre_ref[...] - mean) * inv * g_ref[...] + be_ref[...]
    r = jnp.maximum(bn, 0.0)
    h2 = jnp.dot(r, w_ref[...], preferred_element_type=jnp.float32)
    p2_ref[...] = h2 * _dinv_of(deg_ref[...])


def _tc_b2(pre, s1, s2, deg, g, be, w2):
    return pl.pallas_call(
        _tc_b2_body,
        grid=(NB,),
        in_specs=[
            pl.BlockSpec((BLK, HID), lambda i: (i, 0)),
            pl.BlockSpec((NB, 1, HID), lambda i: (0, 0, 0)),
            pl.BlockSpec((NB, 1, HID), lambda i: (0, 0, 0)),
            pl.BlockSpec((NC, BLK, 16), lambda i: (0, i, 0)),
            pl.BlockSpec((1, HID), lambda i: (0, 0)),
            pl.BlockSpec((1, HID), lambda i: (0, 0)),
            pl.BlockSpec((HID, OUT_CH), lambda i: (0, 0)),
        ],
        out_specs=pl.BlockSpec((BLK, OUT_CH), lambda i: (i, 0)),
        out_shape=jax.ShapeDtypeStruct((NP, OUT_CH), jnp.float32),
    )(pre, s1, s2, deg, g, be, w2)


def _tc_pre2_body(acc_ref, p_ref, deg_ref, b_ref, pre_ref, s1_ref, s2_ref):
    i = pl.program_id(0)
    d = _dinv_of(deg_ref[...])
    a = acc_ref[...]
    pre = (a[0] + a[1] + p_ref[...]) * d + b_ref[...]
    pre_ref[...] = pre
    row = i * BLK + lax.broadcasted_iota(jnp.int32, (BLK, 1), 0)
    pm = jnp.where(row < N, pre, 0.0)
    s1_ref[0, 0] = jnp.sum(pm, axis=0)
    s2_ref[0, 0] = jnp.sum(pm * pm, axis=0)


def _tc_pre2(acc, p, deg, b):
    return pl.pallas_call(
        _tc_pre2_body,
        grid=(NB,),
        in_specs=[
            pl.BlockSpec((NC, BLK, OUT_CH), lambda i: (0, i, 0)),
            pl.BlockSpec((BLK, OUT_CH), lambda i: (i, 0)),
            pl.BlockSpec((NC, BLK, 16), lambda i: (0, i, 0)),
            pl.BlockSpec((1, OUT_CH), lambda i: (0, 0)),
        ],
        out_specs=[
            pl.BlockSpec((BLK, OUT_CH), lambda i: (i, 0)),
            pl.BlockSpec((1, 1, OUT_CH), lambda i: (i, 0, 0)),
            pl.BlockSpec((1, 1, OUT_CH), lambda i: (i, 0, 0)),
        ],
        out_shape=[
            jax.ShapeDtypeStruct((NP, OUT_CH), jnp.float32),
            jax.ShapeDtypeStruct((NB, 1, OUT_CH), jnp.float32),
            jax.ShapeDtypeStruct((NB, 1, OUT_CH), jnp.float32),
        ],
    )(acc, p, deg, b)


def _tc_c2_body(pre_ref, s1_ref, s2_ref, g_ref, be_ref, out_ref):
    mean, inv = _bn_stats(s1_ref, s2_ref)
    out_ref[...] = (pre_ref[...] - mean) * inv * g_ref[...] + be_ref[...]


def _tc_c2(pre, s1, s2, g, be):
    return pl.pallas_call(
        _tc_c2_body,
        grid=(NB,),
        in_specs=[
            pl.BlockSpec((BLK, OUT_CH), lambda i: (i, 0)),
            pl.BlockSpec((NB, 1, OUT_CH), lambda i: (0, 0, 0)),
            pl.BlockSpec((NB, 1, OUT_CH), lambda i: (0, 0, 0)),
            pl.BlockSpec((1, OUT_CH), lambda i: (0, 0)),
            pl.BlockSpec((1, OUT_CH), lambda i: (0, 0)),
        ],
        out_specs=pl.BlockSpec((BLK, OUT_CH), lambda i: (i, 0)),
        out_shape=jax.ShapeDtypeStruct((NP, OUT_CH), jnp.float32),
    )(pre, s1, s2, g, be)


def kernel(x, edge_index, W1, b1, g1, be1, W2, b2, g2, be2):
    src = edge_index[0].astype(jnp.int32)
    dst = edge_index[1].astype(jnp.int32)

    pad_seg = NS * CSEG * K - E
    src_seg = jnp.concatenate([src, jnp.zeros((pad_seg,), jnp.int32)])
    src_seg = src_seg.reshape(NS, CSEG, K)
    src_seg2 = jnp.stack([src_seg, src_seg + NP])
    dst_seg = jnp.concatenate(
        [dst, jnp.full((pad_seg,), DUMMY, jnp.int32)]
    ).reshape(NS, CSEG, K)

    pad_deg = NC * NS * CDEG * K - E
    dst_deg_flat = jnp.concatenate(
        [dst, jnp.full((pad_deg,), DUMMY, jnp.int32)]
    ).reshape(NC, NS, CDEG * K)
    pad_es = NC * NS * CES * K - E
    dst_es = jnp.concatenate(
        [dst, jnp.full((pad_es,), DUMMY, jnp.int32)]
    ).reshape(NC, NS, CES, K)
    src_es = jnp.concatenate([src, jnp.zeros((pad_es,), jnp.int32)]).reshape(
        NC, NS, CES, K
    )

    xp = jnp.zeros((NP, IN_CH), jnp.float32).at[:N].set(x)

    deg1 = _deg_kernel(dst_deg_flat)
    deg = jnp.broadcast_to(deg1[:, :, None], (NC, NP, 16))

    p1 = _tc_a(deg, xp, W1)
    acc1 = _segsum128(p1.reshape(NC * NP, HID // 2), src_seg2, dst_seg)
    pre1, s1a, s2a = _tc_pre(acc1, p1, deg, b1.reshape(1, HID), HID // 2)
    p2 = _tc_b2(pre1, s1a, s2a, deg, g1.reshape(1, HID), be1.reshape(1, HID), W2)
    acc2 = _segsum_es(p2, src_es, dst_es)
    pre2, s1b, s2b = _tc_pre2(acc2, p2, deg, b2.reshape(1, OUT_CH))
    out = _tc_c2(pre2, s1b, s2b, g2.reshape(1, OUT_CH), be2.reshape(1, OUT_CH))
    return out[:N]


# speedup vs baseline: 1.3165x; 1.3165x over previous
import functools

import jax
import jax.numpy as jnp
from jax import lax
from jax.experimental import pallas as pl
from jax.experimental.pallas import tpu as pltpu
from jax.experimental.pallas import tpu_sc as plsc

N = 10000
E = 320000
IN_CH = 128
HID = 256
OUT_CH = 128

NC = 2
NS = 16
NP = 10240
DUMMY = N
K = 128
CSEG = 160
CHP = 80
CDEG = 79
NPS = 10112
TROWS = 632
BLK = 1280
NB = NP // BLK

_WB_K = ((0, 128), (128, 128), (256, 128), (384, 128), (512, 120))
_WB_ZB = ((0, 320), (320, 312))

_mesh = plsc.VectorSubcoreMesh(core_axis_name="c", subcore_axis_name="s")


@functools.partial(
    pl.kernel,
    out_type=jax.ShapeDtypeStruct((NC, NP), jnp.float32),
    mesh=_mesh,
    compiler_params=pltpu.CompilerParams(needs_layout_passes=False),
    scratch_types=[
        pltpu.VMEM((CDEG * K,), jnp.int32),
        pltpu.VMEM((NP,), jnp.float32),
        pltpu.VMEM((NS, NP // NS), jnp.float32),
        pltpu.VMEM((NP // NS,), jnp.float32),
        pltpu.VMEM_SHARED((NS, NP), jnp.float32),
    ],
)
def _deg_kernel(dst_hbm, deg_hbm, idx_v, cnt_v, red_v, out_v, spm):
    cid = lax.axis_index("c")
    sid = lax.axis_index("s")
    rpt = NP // NS
    pltpu.sync_copy(dst_hbm.at[cid, sid], idx_v)
    zero16 = jnp.zeros((16,), jnp.float32)
    ones16 = jnp.ones((16,), jnp.float32)

    def _z(i, _):
        cnt_v[pl.ds(i * 16, 16)] = zero16
        return 0

    lax.fori_loop(0, NP // 16, _z, 0)

    def _count(i, _):
        idx16 = idx_v[pl.ds(i * 16, 16)]
        plsc.addupdate_scatter(cnt_v, [idx16], ones16)
        return 0

    lax.fori_loop(0, CDEG * K // 16, _count, 0)
    pltpu.sync_copy(cnt_v, spm.at[sid])
    plsc.subcore_barrier()
    for t in range(NS):
        pltpu.sync_copy(spm.at[t, pl.ds(sid * rpt, rpt)], red_v.at[t])

    def _red(g, _):
        acc = jnp.zeros((16,), jnp.float32)
        for t in range(NS):
            acc = acc + red_v[t, pl.ds(g * 16, 16)]
        out_v[pl.ds(g * 16, 16)] = acc
        return 0

    lax.fori_loop(0, rpt // 16, _red, 0)
    pltpu.sync_copy(out_v, deg_hbm.at[cid, pl.ds(sid * rpt, rpt)])


def _make_segsum(ch):
    @functools.partial(
        pl.kernel,
        out_type=jax.ShapeDtypeStruct((NC, NP, ch), jnp.float32),
        mesh=_mesh,
        scratch_types=[
            pltpu.VMEM((CHP, K), jnp.int32),
            pltpu.VMEM((CHP, K), jnp.int32),
            pltpu.VMEM((K, ch), jnp.float32),
            pltpu.VMEM_SHARED((NPS, ch), jnp.float32),
            pltpu.SemaphoreType.DMA,
        ],
    )
    def _segsum(p_hbm, src_hbm, dst_hbm, acc_hbm, srcv, dstv, gbuf, acc_s, sem):
        cid = lax.axis_index("c")
        sid = lax.axis_index("s")
        base = sid * TROWS

        def _fill(i, _):
            for c in range(ch // 16):
                gbuf[i, pl.ds(c * 16, 16)] = jnp.zeros((16,), jnp.float32)
            return 0

        lax.fori_loop(0, K, _fill, 0)
        for off, n in _WB_K:
            pltpu.sync_copy(gbuf.at[pl.ds(0, n)], acc_s.at[pl.ds(base + off, n)])
        plsc.subcore_barrier()

        for p0 in range(0, CSEG, CHP):
            pltpu.sync_copy(src_hbm.at[cid, sid, pl.ds(p0, CHP)], srcv)
            pltpu.sync_copy(dst_hbm.at[sid, pl.ds(p0, CHP)], dstv)

            def _chunk(j, _):
                pltpu.async_copy(p_hbm.at[srcv.at[j]], gbuf, sem).wait()
                pltpu.sync_copy(gbuf, acc_s.at[dstv.at[j]], add=True)
                return 0

            lax.fori_loop(0, CHP, _chunk, 0)

        plsc.subcore_barrier()
        for off, n in _WB_K:
            pltpu.sync_copy(acc_s.at[pl.ds(base + off, n)], gbuf.at[pl.ds(0, n)])
            pltpu.sync_copy(gbuf.at[pl.ds(0, n)], acc_hbm.at[cid, pl.ds(base + off, n)])

    return _segsum


_segsum128 = _make_segsum(128)


@functools.partial(
    pl.kernel,
    out_type=jax.ShapeDtypeStruct((NC, NP, OUT_CH), jnp.float32),
    mesh=_mesh,
    scratch_types=[
        pltpu.VMEM((CDEG, K), jnp.int32),
        pltpu.VMEM((CDEG, K), jnp.int32),
        pltpu.VMEM((K, OUT_CH), jnp.float32),
        pltpu.VMEM_SHARED((NPS, OUT_CH), jnp.float32),
        pltpu.SemaphoreType.DMA,
    ],
)
def _segsum_es(p_hbm, src_hbm, dst_hbm, acc_hbm, srcv, dstv, gbuf, acc_s, sem):
    cid = lax.axis_index("c")
    sid = lax.axis_index("s")
    base = sid * TROWS
    pltpu.sync_copy(src_hbm.at[cid, sid], srcv)
    pltpu.sync_copy(dst_hbm.at[cid, sid], dstv)

    def _fill(i, _):
        for c in range(OUT_CH // 16):
            gbuf[i, pl.ds(c * 16, 16)] = jnp.zeros((16,), jnp.float32)
        return 0

    lax.fori_loop(0, K, _fill, 0)
    for off, n in _WB_K:
        pltpu.sync_copy(gbuf.at[pl.ds(0, n)], acc_s.at[pl.ds(base + off, n)])
    plsc.subcore_barrier()

    def _chunk(j, _):
        pltpu.async_copy(p_hbm.at[srcv.at[j]], gbuf, sem).wait()
        pltpu.sync_copy(gbuf, acc_s.at[dstv.at[j]], add=True)
        return 0

    lax.fori_loop(0, CDEG, _chunk, 0)
    plsc.subcore_barrier()
    for off, n in _WB_K:
        pltpu.sync_copy(acc_s.at[pl.ds(base + off, n)], gbuf.at[pl.ds(0, n)])
        pltpu.sync_copy(gbuf.at[pl.ds(0, n)], acc_hbm.at[cid, pl.ds(base + off, n)])


def _dinv_of(deg_blk):
    return lax.rsqrt(deg_blk[0, :, 0:1] + deg_blk[1, :, 0:1] + 1.0)


def _tc_a_body(deg_ref, x_ref, w_ref, p_ref):
    d = _dinv_of(deg_ref[...])
    h = jnp.dot(x_ref[...], w_ref[...], preferred_element_type=jnp.float32)
    p_ref[0] = h * d


def _tc_a(deg, xp, w1):
    return pl.pallas_call(
        _tc_a_body,
        grid=(2, NB),
        in_specs=[
            pl.BlockSpec((NC, BLK, 16), lambda ph, i: (0, i, 0)),
            pl.BlockSpec((BLK, IN_CH), lambda ph, i: (i, 0)),
            pl.BlockSpec((IN_CH, HID // 2), lambda ph, i: (0, ph)),
        ],
        out_specs=pl.BlockSpec((1, BLK, HID // 2), lambda ph, i: (ph, i, 0)),
        out_shape=jax.ShapeDtypeStruct((2, NP, HID // 2), jnp.float32),
    )(deg, xp, w1)


def _make_pre_body(ch):
    def _body(acc_ref, p_ref, deg_ref, b_ref, pre_ref, s1_ref, s2_ref):
        i = pl.program_id(0)
        d = _dinv_of(deg_ref[...])
        a = acc_ref[...]
        p = p_ref[...]
        u0 = (a[0] + p[0]) * d
        u1 = (a[1] + p[1]) * d
        pre = jnp.concatenate([u0, u1], axis=1) + b_ref[...]
        pre_ref[...] = pre
        row = i * BLK + lax.broadcasted_iota(jnp.int32, (BLK, 1), 0)
        pm = jnp.where(row < N, pre, 0.0)
        s1_ref[0, 0] = jnp.sum(pm, axis=0)
        s2_ref[0, 0] = jnp.sum(pm * pm, axis=0)

    return _body


def _tc_pre(acc, p, deg, b, ch):
    return pl.pallas_call(
        _make_pre_body(ch),
        grid=(NB,),
        in_specs=[
            pl.BlockSpec((NC, BLK, ch), lambda i: (0, i, 0)),
            pl.BlockSpec((NC, BLK, ch), lambda i: (0, i, 0)),
            pl.BlockSpec((NC, BLK, 16), lambda i: (0, i, 0)),
            pl.BlockSpec((1, 2 * ch), lambda i: (0, 0)),
        ],
        out_specs=[
            pl.BlockSpec((BLK, 2 * ch), lambda i: (i, 0)),
            pl.BlockSpec((1, 1, 2 * ch), lambda i: (i, 0, 0)),
            pl.BlockSpec((1, 1, 2 * ch), lambda i: (i, 0, 0)),
        ],
        out_shape=[
            jax.ShapeDtypeStruct((NP, 2 * ch), jnp.float32),
            jax.ShapeDtypeStruct((NB, 1, 2 * ch), jnp.float32),
            jax.ShapeDtypeStruct((NB, 1, 2 * ch), jnp.float32),
        ],
    )(acc, p, deg, b)


def _bn_stats(s1_ref, s2_ref):
    s1 = jnp.sum(s1_ref[...][:, 0, :], axis=0, keepdims=True)
    s2 = jnp.sum(s2_ref[...][:, 0, :], axis=0, keepdims=True)
    mean = s1 / N
    var = s2 / N - mean * mean
    inv = lax.rsqrt(var + 1e-5)
    return mean, inv


def _tc_b2_body(pre_ref, s1_ref, s2_ref, deg_ref, g_ref, be_ref, w_ref, p2_ref):
    mean, inv = _bn_stats(s1_ref, s2_ref)
    bn = (pre_ref[...] - mean) * inv * g_ref[...] + be_ref[...]
    r = jnp.maximum(bn, 0.0)
    h2 = jnp.dot(r, w_ref[...], preferred_element_type=jnp.float32)
    p2_ref[...] = h2 * _dinv_of(deg_ref[...])


def _tc_b2(pre, s1, s2, deg, g, be, w2):
    return pl.pallas_call(
        _tc_b2_body,
        grid=(NB,),
        in_specs=[
            pl.BlockSpec((BLK, HID), lambda i: (i, 0)),
            pl.BlockSpec((NB, 1, HID), lambda i: (0, 0, 0)),
            pl.BlockSpec((NB, 1, HID), lambda i: (0, 0, 0)),
            pl.BlockSpec((NC, BLK, 16), lambda i: (0, i, 0)),
            pl.BlockSpec((1, HID), lambda i: (0, 0)),
            pl.BlockSpec((1, HID), lambda i: (0, 0)),
            pl.BlockSpec((HID, OUT_CH), lambda i: (0, 0)),
        ],
        out_specs=pl.BlockSpec((BLK, OUT_CH), lambda i: (i, 0)),
        out_shape=jax.ShapeDtypeStruct((NP, OUT_CH), jnp.float32),
    )(pre, s1, s2, deg, g, be, w2)


def _tc_pre2_body(acc_ref, p_ref, deg_ref, b_ref, pre_ref, s1_ref, s2_ref):
    i = pl.program_id(0)
    d = _dinv_of(deg_ref[...])
    a = acc_ref[...]
    pre = (a[0] + a[1] + p_ref[...]) * d + b_ref[...]
    pre_ref[...] = pre
    row = i * BLK + lax.broadcasted_iota(jnp.int32, (BLK, 1), 0)
    pm = jnp.where(row < N, pre, 0.0)
    s1_ref[0, 0] = jnp.sum(pm, axis=0)
    s2_ref[0, 0] = jnp.sum(pm * pm, axis=0)


def _tc_pre2(acc, p, deg, b):
    return pl.pallas_call(
        _tc_pre2_body,
        grid=(NB,),
        in_specs=[
            pl.BlockSpec((NC, BLK, OUT_CH), lambda i: (0, i, 0)),
            pl.BlockSpec((BLK, OUT_CH), lambda i: (i, 0)),
            pl.BlockSpec((NC, BLK, 16), lambda i: (0, i, 0)),
            pl.BlockSpec((1, OUT_CH), lambda i: (0, 0)),
        ],
        out_specs=[
            pl.BlockSpec((BLK, OUT_CH), lambda i: (i, 0)),
            pl.BlockSpec((1, 1, OUT_CH), lambda i: (i, 0, 0)),
            pl.BlockSpec((1, 1, OUT_CH), lambda i: (i, 0, 0)),
        ],
        out_shape=[
            jax.ShapeDtypeStruct((NP, OUT_CH), jnp.float32),
            jax.ShapeDtypeStruct((NB, 1, OUT_CH), jnp.float32),
            jax.ShapeDtypeStruct((NB, 1, OUT_CH), jnp.float32),
        ],
    )(acc, p, deg, b)


def _tc_c2_body(pre_ref, s1_ref, s2_ref, g_ref, be_ref, out_ref):
    mean, inv = _bn_stats(s1_ref, s2_ref)
    out_ref[...] = (pre_ref[...] - mean) * inv * g_ref[...] + be_ref[...]


def _tc_c2(pre, s1, s2, g, be):
    return pl.pallas_call(
        _tc_c2_body,
        grid=(NB,),
        in_specs=[
            pl.BlockSpec((BLK, OUT_CH), lambda i: (i, 0)),
            pl.BlockSpec((NB, 1, OUT_CH), lambda i: (0, 0, 0)),
            pl.BlockSpec((NB, 1, OUT_CH), lambda i: (0, 0, 0)),
            pl.BlockSpec((1, OUT_CH), lambda i: (0, 0)),
            pl.BlockSpec((1, OUT_CH), lambda i: (0, 0)),
        ],
        out_specs=pl.BlockSpec((BLK, OUT_CH), lambda i: (i, 0)),
        out_shape=jax.ShapeDtypeStruct((NP, OUT_CH), jnp.float32),
    )(pre, s1, s2, g, be)


def kernel(x, edge_index, W1, b1, g1, be1, W2, b2, g2, be2):
    src = edge_index[0].astype(jnp.int32)
    dst = edge_index[1].astype(jnp.int32)

    pad_seg = NS * CSEG * K - E
    src_seg = jnp.concatenate([src, jnp.zeros((pad_seg,), jnp.int32)])
    src_seg = src_seg.reshape(NS, CSEG, K)
    src_seg2 = jnp.stack([src_seg, src_seg + NP])
    dst_seg = jnp.concatenate(
        [dst, jnp.full((pad_seg,), DUMMY, jnp.int32)]
    ).reshape(NS, CSEG, K)

    pad_deg = NC * NS * CDEG * K - E
    dst_deg = jnp.concatenate(
        [dst, jnp.full((pad_deg,), DUMMY, jnp.int32)]
    ).reshape(NC, NS, CDEG, K)
    dst_deg_flat = dst_deg.reshape(NC, NS, CDEG * K)
    src_es = jnp.concatenate([src, jnp.zeros((pad_deg,), jnp.int32)]).reshape(
        NC, NS, CDEG, K
    )

    xp = jnp.zeros((NP, IN_CH), jnp.float32).at[:N].set(x)

    deg1 = _deg_kernel(dst_deg_flat)
    deg = jnp.broadcast_to(deg1[:, :, None], (NC, NP, 16))

    p1 = _tc_a(deg, xp, W1)
    acc1 = _segsum128(p1.reshape(NC * NP, HID // 2), src_seg2, dst_seg)
    pre1, s1a, s2a = _tc_pre(acc1, p1, deg, b1.reshape(1, HID), HID // 2)
    p2 = _tc_b2(pre1, s1a, s2a, deg, g1.reshape(1, HID), be1.reshape(1, HID), W2)
    acc2 = _segsum_es(p2, src_es, dst_deg)
    pre2, s1b, s2b = _tc_pre2(acc2, p2, deg, b2.reshape(1, OUT_CH))
    out = _tc_c2(pre2, s1b, s2b, g2.reshape(1, OUT_CH), be2.reshape(1, OUT_CH))
    return out[:N]
